# baseline (speedup 1.0000x reference)
import functools

import jax
import jax.numpy as jnp
from jax import lax
from jax.experimental import pallas as pl
from jax.experimental.pallas import tpu as pltpu
from jax.experimental.pallas import tpu_sc as plsc

B = 16384
D = 16
NC, NS = 2, 16
NW = NC * NS
BPW = B // NW
GC = 128
NCHUNK = BPW // GC

@functools.cache
def _make_sc_gather():
    mesh = plsc.VectorSubcoreMesh(
        core_axis_name="c", subcore_axis_name="s", num_cores=NC, num_subcores=NS
    )

    @functools.partial(
        pl.kernel,
        out_type=jax.ShapeDtypeStruct((B, D), jnp.float32),
        mesh=mesh,
        scratch_types=[
            pltpu.VMEM((NCHUNK, GC), jnp.int32),
            pltpu.VMEM((BPW, D), jnp.float32),
            pltpu.SemaphoreType.DMA,
        ],
        compiler_params=pltpu.CompilerParams(use_tc_tiling_on_sc=False),
    )
    def _sc_gather(idx_hbm, table_hbm, out_hbm, idx_v, rows_v, sem):
        wid = lax.axis_index("s") * NC + lax.axis_index("c")
        base = wid * NCHUNK
        pltpu.sync_copy(idx_hbm.at[pl.ds(base, NCHUNK)], idx_v)
        copies = []
        for j in range(NCHUNK):
            copies.append(
                pltpu.async_copy(
                    table_hbm.at[idx_v.at[j]], rows_v.at[pl.ds(j * GC, GC)], sem
                )
            )
        for c in copies:
            c.wait()
        pltpu.sync_copy(rows_v, out_hbm.at[pl.ds(wid * BPW, BPW)])

    return _sc_gather


def _mlp_body(year_ref, e_ref, w1y_ref, w1e_ref, b1_ref, w2_ref, b2_ref,
              w3_ref, b3_ref, out_ref):
    e = e_ref[...]
    year = year_ref[...]
    h1 = lax.dot_general(e, w1e_ref[...], (((1,), (1,)), ((), ())),
                         preferred_element_type=jnp.float32)
    h1y = lax.dot_general(year, w1y_ref[...], (((1,), (0,)), ((), ())),
                          preferred_element_type=jnp.float32)
    h1 = jnp.maximum(h1 + h1y + b1_ref[...], 0.0)
    h2 = lax.dot_general(h1, w2_ref[...], (((1,), (1,)), ((), ())),
                         preferred_element_type=jnp.float32)
    h2 = jnp.maximum(h2 + b2_ref[...], 0.0)
    out = lax.dot_general(h2, w3_ref[...], (((1,), (1,)), ((), ())),
                          preferred_element_type=jnp.float32)
    out_ref[...] = out[:, :1] + b3_ref[0]


BK = 4096


def _mlp(year, e, w1y, w1e, b1, w2, b2, w3, b3):
    full = lambda s: pl.BlockSpec(s, lambda i: (0, 0))
    return pl.pallas_call(
        _mlp_body,
        grid=(B // BK,),
        in_specs=[
            pl.BlockSpec((BK, 1), lambda i: (i, 0)),
            pl.BlockSpec((BK, D), lambda i: (i, 0)),
            full((1, 32)),
            full((32, D)),
            full((1, 32)),
            full((32, 32)),
            full((1, 32)),
            full((8, 32)),
            pl.BlockSpec(memory_space=pltpu.SMEM),
        ],
        out_specs=pl.BlockSpec((BK, 1), lambda i: (i, 0)),
        out_shape=jax.ShapeDtypeStruct((B, 1), jnp.float32),
    )(year, e, w1y, w1e, b1, w2, b2, w3, b3)


def kernel(x, embed, W1, b1, W2, b2, W3, b3):
    idx = x[:, 0].astype(jnp.int32).reshape(NW * NCHUNK, GC)
    year = x[:, 1:2]
    e = _make_sc_gather()(idx, embed)
    w1y = W1[:, 0].reshape(1, 32)
    w1e = W1[:, 1:]
    w3p = jnp.zeros((8, 32), jnp.float32).at[0].set(W3[0])
    return _mlp(year, e, w1y, w1e, b1.reshape(1, 32), W2, b2.reshape(1, 32),
                w3p, b3)
